# Initial kernel scaffold; baseline (speedup 1.0000x reference)
import jax
import jax.numpy as jnp
from jax.experimental import pallas as pl


def kernel(x, genre_embed, rms_w, wg_W, wg_b, gg_W, gg_b, eW1, eb1, eW2, eb2, eW3, eb3):
    raise NotImplementedError("write your pallas kernel here")


# speedup vs baseline: 1.3304x; 1.3304x over previous
import jax
import jax.numpy as jnp
from jax.experimental import pallas as pl
from jax.experimental.pallas import tpu as pltpu

EPS = 1e-6


def _dotT(a, b):
    return jax.lax.dot_general(a, b, (((1,), (1,)), ((), ())),
                               preferred_element_type=jnp.float32)


def _gate_kernel(x_ref, ge_ref, rmsw_ref, wgW_ref, wgb_ref, ggW_ref, ggb_ref,
                 coef_ref):
    x = x_ref[...]
    var = jnp.mean(x * x, axis=-1, keepdims=True)
    xn = x * jax.lax.rsqrt(var + EPS) * rmsw_ref[...]
    gate = _dotT(xn, wgW_ref[...]) + wgb_ref[...]
    gate = gate + _dotT(ge_ref[...], ggW_ref[...]) + ggb_ref[...]
    gate = gate - jnp.max(gate, axis=-1, keepdims=True)
    eg = jnp.exp(gate)
    p = eg / jnp.sum(eg, axis=-1, keepdims=True)
    E = p.shape[-1]
    iota = jax.lax.broadcasted_iota(jnp.int32, p.shape, 1)
    w1 = jnp.max(p, axis=-1, keepdims=True)
    i1 = jnp.min(jnp.where(p == w1, iota, E), axis=-1, keepdims=True)
    oh1 = iota == i1
    p2 = jnp.where(oh1, -jnp.inf, p)
    w2 = jnp.max(p2, axis=-1, keepdims=True)
    i2 = jnp.min(jnp.where(p2 == w2, iota, E), axis=-1, keepdims=True)
    oh2 = iota == i2
    coef_ref[...] = jnp.where(oh1, w1, 0.0) + jnp.where(oh2, w2, 0.0)


def _h1_kernel(x_ref, w1_ref, b1_ref, h1_ref):
    h1_ref[0] = jnp.maximum(_dotT(x_ref[...], w1_ref[0]) + b1_ref[0], 0.0)


def _h2_kernel(h1_ref, w2_ref, b2_ref, h2_ref):
    h2_ref[0] = jnp.maximum(_dotT(h1_ref[0], w2_ref[0]) + b2_ref[0], 0.0)


def _out_kernel(h2_ref, w3_ref, b3_ref, coef_ref, o_ref):
    e = pl.program_id(1)
    E = coef_ref.shape[-1]
    lane = jax.lax.broadcasted_iota(jnp.int32, coef_ref.shape, 1)
    c = jnp.sum(jnp.where(lane == e, coef_ref[...], 0.0), axis=-1,
                keepdims=True)
    y = _dotT(h2_ref[0] * c, w3_ref[0]) + c * b3_ref[0]

    @pl.when(e == 0)
    def _():
        o_ref[...] = y

    @pl.when(e != 0)
    def _():
        o_ref[...] += y


def kernel(x, genre_embed, rms_w, wg_W, wg_b, gg_W, gg_b,
           eW1, eb1, eW2, eb2, eW3, eb3):
    b, s, h = x.shape
    S = b * s
    E, M, H = eW1.shape
    G = gg_W.shape[1]
    x2 = x.reshape(S, H)
    ge2 = genre_embed.reshape(S, G)

    coef = pl.pallas_call(
        _gate_kernel,
        out_shape=jax.ShapeDtypeStruct((S, E), jnp.float32),
    )(x2, ge2, rms_w.reshape(1, H), wg_W, wg_b.reshape(1, E),
      gg_W, gg_b.reshape(1, E))

    BM = 512
    h1 = pl.pallas_call(
        _h1_kernel,
        grid=(E, M // BM),
        in_specs=[
            pl.BlockSpec((S, H), lambda e, m: (0, 0)),
            pl.BlockSpec((1, BM, H), lambda e, m: (e, m, 0)),
            pl.BlockSpec((1, 1, BM), lambda e, m: (e, 0, m)),
        ],
        out_specs=pl.BlockSpec((1, S, BM), lambda e, m: (e, 0, m)),
        out_shape=jax.ShapeDtypeStruct((E, S, M), jnp.float32),
    )(x2, eW1, eb1.reshape(E, 1, M))

    BN = 512
    h2 = pl.pallas_call(
        _h2_kernel,
        grid=(E, M // BN),
        in_specs=[
            pl.BlockSpec((1, S, M), lambda e, n: (e, 0, 0)),
            pl.BlockSpec((1, BN, M), lambda e, n: (e, n, 0)),
            pl.BlockSpec((1, 1, BN), lambda e, n: (e, 0, n)),
        ],
        out_specs=pl.BlockSpec((1, S, BN), lambda e, n: (e, 0, n)),
        out_shape=jax.ShapeDtypeStruct((E, S, M), jnp.float32),
    )(h1, eW2, eb2.reshape(E, 1, M))

    TS = 1024
    out = pl.pallas_call(
        _out_kernel,
        grid=(S // TS, E),
        in_specs=[
            pl.BlockSpec((1, TS, M), lambda t, e: (e, t, 0)),
            pl.BlockSpec((1, H, M), lambda t, e: (e, 0, 0)),
            pl.BlockSpec((1, 1, H), lambda t, e: (e, 0, 0)),
            pl.BlockSpec((TS, E), lambda t, e: (t, 0)),
        ],
        out_specs=pl.BlockSpec((TS, H), lambda t, e: (t, 0)),
        out_shape=jax.ShapeDtypeStruct((S, H), jnp.float32),
    )(h2, eW3, eb3.reshape(E, 1, H), coef)

    return out.reshape(b, s, h)


# speedup vs baseline: 1.7396x; 1.3075x over previous
import functools

import jax
import jax.numpy as jnp
from jax import lax
from jax.experimental import pallas as pl
from jax.experimental.pallas import tpu as pltpu
from jax.experimental.pallas import tpu_sc as plsc

EPS = 1e-6
TOPK = 2
BT = 256


def _dotT(a, b):
    return jax.lax.dot_general(a, b, (((1,), (1,)), ((), ())),
                               preferred_element_type=jnp.float32)


def _route_kernel(x_ref, ge_ref, rmsw_ref, wgW_ref, wgb_ref, ggW_ref, ggb_ref,
                  pos1_ref, pos2_ref, w1_ref, w2_ref, be_ref, bv_ref):
    S = x_ref.shape[0]
    NB = be_ref.shape[0]
    x = x_ref[...]
    var = jnp.mean(x * x, axis=-1, keepdims=True)
    xn = x * jax.lax.rsqrt(var + EPS) * rmsw_ref[...]
    gate = _dotT(xn, wgW_ref[...]) + wgb_ref[...]
    gate = gate + _dotT(ge_ref[...], ggW_ref[...]) + ggb_ref[...]
    gate = gate - jnp.max(gate, axis=-1, keepdims=True)
    eg = jnp.exp(gate)
    p = eg / jnp.sum(eg, axis=-1, keepdims=True)
    E = p.shape[-1]
    iota = jax.lax.broadcasted_iota(jnp.int32, p.shape, 1)
    w1 = jnp.max(p, axis=-1, keepdims=True)
    i1 = jnp.min(jnp.where(p == w1, iota, E), axis=-1, keepdims=True)
    oh1 = iota == i1
    pm = jnp.where(oh1, -jnp.inf, p)
    w2 = jnp.max(pm, axis=-1, keepdims=True)
    i2 = jnp.min(jnp.where(pm == w2, iota, E), axis=-1, keepdims=True)
    oh2 = iota == i2
    w1_ref[...] = jnp.broadcast_to(w1, (S, 16))
    w2_ref[...] = jnp.broadcast_to(w2, (S, 16))

    tot = (oh1 | oh2).astype(jnp.float32)
    r = jax.lax.broadcasted_iota(jnp.int32, (S, S), 0)
    c = jax.lax.broadcasted_iota(jnp.int32, (S, S), 1)
    L = (r > c).astype(jnp.float32)
    cum = jax.lax.dot_general(L, tot, (((1,), (0,)), ((), ())),
                              preferred_element_type=jnp.float32)
    counts = jnp.sum(tot, axis=0, keepdims=True)
    padded = jnp.ceil(counts / BT) * BT
    re = jax.lax.broadcasted_iota(jnp.int32, (E, E), 0)
    ce = jax.lax.broadcasted_iota(jnp.int32, (E, E), 1)
    offs = jax.lax.dot_general(padded, (re < ce).astype(jnp.float32),
                               (((1,), (0,)), ((), ())),
                               preferred_element_type=jnp.float32)
    pos1 = jnp.sum(jnp.where(oh1, offs + cum, 0.0), axis=1, keepdims=True)
    pos2 = jnp.sum(jnp.where(oh2, offs + cum, 0.0), axis=1, keepdims=True)
    pos1_ref[...] = pos1.astype(jnp.int32)
    pos2_ref[...] = pos2.astype(jnp.int32)

    si = jax.lax.broadcasted_iota(jnp.int32, (NB, E), 0).astype(jnp.float32) * BT
    ee = jax.lax.broadcasted_iota(jnp.int32, (NB, E), 1)
    inseg = (si >= offs) & (si < offs + padded)
    bv = jnp.sum(inseg.astype(jnp.int32), axis=1, keepdims=True)
    be = jnp.sum(jnp.where(inseg, ee, 0), axis=1, keepdims=True)
    be_ref[...] = jnp.where(bv > 0, be, E - 1)
    bv_ref[...] = bv


def _ffn1_kernel(be_ref, bv_ref, xs_ref, w_ref, b_ref, o_ref):
    i = pl.program_id(0)

    @pl.when(bv_ref[i] > 0)
    def _():
        o_ref[...] = jnp.maximum(_dotT(xs_ref[...], w_ref[0]) + b_ref[0], 0.0)


def _ffn2_kernel(be_ref, bv_ref, h1_ref, w_ref, b_ref, o_ref):
    i = pl.program_id(0)

    @pl.when(bv_ref[i] > 0)
    def _():
        o_ref[...] = jnp.maximum(_dotT(h1_ref[...], w_ref[0]) + b_ref[0], 0.0)


def _ffn3_kernel(be_ref, bv_ref, h2_ref, w_ref, b_ref, o_ref):
    i = pl.program_id(0)

    @pl.when(bv_ref[i] > 0)
    def _():
        o_ref[...] = _dotT(h2_ref[...], w_ref[0]) + b_ref[0]


def _make_dispatch(S, H, P, NC, NS):
    NW = NC * NS
    CHUNK = S // NW
    mesh = plsc.VectorSubcoreMesh(core_axis_name="c", subcore_axis_name="s")

    @functools.partial(
        pl.kernel, mesh=mesh,
        out_type=jax.ShapeDtypeStruct((P, H), jnp.float32),
        scratch_types=[
            pltpu.VMEM((CHUNK,), jnp.int32),
            pltpu.VMEM((CHUNK,), jnp.int32),
            pltpu.VMEM((CHUNK, H), jnp.float32),
            pltpu.SemaphoreType.DMA,
        ],
    )
    def dispatch(x_hbm, pos1_hbm, pos2_hbm, xs_hbm, idx1_v, idx2_v, rows_v,
                 sem):
        wid = lax.axis_index("s") * NC + lax.axis_index("c")
        base = wid * CHUNK
        pltpu.sync_copy(pos1_hbm.at[pl.ds(base, CHUNK)], idx1_v)
        pltpu.sync_copy(pos2_hbm.at[pl.ds(base, CHUNK)], idx2_v)
        pltpu.sync_copy(x_hbm.at[pl.ds(base, CHUNK)], rows_v)
        pltpu.async_copy(rows_v, xs_hbm.at[idx1_v], sem).wait()
        pltpu.async_copy(rows_v, xs_hbm.at[idx2_v], sem).wait()

    return dispatch


def _make_combine(S, H, NC, NS):
    NW = NC * NS
    NSUB = 2
    CH = S // NW // NSUB
    NV = H // 16
    mesh = plsc.VectorSubcoreMesh(core_axis_name="c", subcore_axis_name="s")

    @functools.partial(
        pl.kernel, mesh=mesh,
        out_type=jax.ShapeDtypeStruct((S, H), jnp.float32),
        scratch_types=[
            pltpu.VMEM((CH,), jnp.int32),
            pltpu.VMEM((CH,), jnp.int32),
            pltpu.VMEM((CH, 16), jnp.float32),
            pltpu.VMEM((CH, 16), jnp.float32),
            pltpu.VMEM((CH, H), jnp.float32),
            pltpu.VMEM((CH, H), jnp.float32),
            pltpu.SemaphoreType.DMA,
        ],
    )
    def combine(ys_hbm, pos1_hbm, pos2_hbm, w1_hbm, w2_hbm, out_hbm,
                i1_v, i2_v, w1_v, w2_v, r1_v, r2_v, sem):
        wid = lax.axis_index("s") * NC + lax.axis_index("c")
        for cidx in range(NSUB):
            base = (wid * NSUB + cidx) * CH
            pltpu.sync_copy(pos1_hbm.at[pl.ds(base, CH)], i1_v)
            pltpu.sync_copy(pos2_hbm.at[pl.ds(base, CH)], i2_v)
            pltpu.sync_copy(w1_hbm.at[pl.ds(base, CH)], w1_v)
            pltpu.sync_copy(w2_hbm.at[pl.ds(base, CH)], w2_v)
            pltpu.async_copy(ys_hbm.at[i1_v], r1_v, sem).wait()
            pltpu.async_copy(ys_hbm.at[i2_v], r2_v, sem).wait()

            def body(i, _):
                w1s = w1_v[i, :]
                w2s = w2_v[i, :]
                for v in range(NV):
                    sl = pl.ds(v * 16, 16)
                    r1_v[i, sl] = r1_v[i, sl] * w1s + r2_v[i, sl] * w2s
                return 0

            lax.fori_loop(0, CH, body, 0)
            pltpu.sync_copy(r1_v, out_hbm.at[pl.ds(base, CH)])

    return combine


def kernel(x, genre_embed, rms_w, wg_W, wg_b, gg_W, gg_b,
           eW1, eb1, eW2, eb2, eW3, eb3):
    b, s, h = x.shape
    S = b * s
    E, M, H = eW1.shape
    G = gg_W.shape[1]
    P = TOPK * S + E * BT
    NB = P // BT
    x2 = x.reshape(S, H)
    ge2 = genre_embed.reshape(S, G)

    f32 = jnp.float32
    i32 = jnp.int32
    pos1, pos2, w1, w2, be, bv = pl.pallas_call(
        _route_kernel,
        out_shape=(
            jax.ShapeDtypeStruct((S, 1), i32),
            jax.ShapeDtypeStruct((S, 1), i32),
            jax.ShapeDtypeStruct((S, 16), f32),
            jax.ShapeDtypeStruct((S, 16), f32),
            jax.ShapeDtypeStruct((NB, 1), i32),
            jax.ShapeDtypeStruct((NB, 1), i32),
        ),
    )(x2, ge2, rms_w.reshape(1, H), wg_W, wg_b.reshape(1, E),
      gg_W, gg_b.reshape(1, E))
    pos1 = pos1.reshape(S)
    pos2 = pos2.reshape(S)
    be = be.reshape(NB)
    bv = bv.reshape(NB)

    info = plsc.get_sparse_core_info()
    NC, NS = info.num_cores, info.num_subcores

    xs = _make_dispatch(S, H, P, NC, NS)(x2, pos1, pos2)

    h1 = pl.pallas_call(
        _ffn1_kernel,
        grid_spec=pltpu.PrefetchScalarGridSpec(
            num_scalar_prefetch=2,
            grid=(NB,),
            in_specs=[
                pl.BlockSpec((BT, H), lambda i, be, bv: (i, 0)),
                pl.BlockSpec((1, M, H), lambda i, be, bv: (be[i], 0, 0)),
                pl.BlockSpec((1, 1, M), lambda i, be, bv: (be[i], 0, 0)),
            ],
            out_specs=pl.BlockSpec((BT, M), lambda i, be, bv: (i, 0)),
        ),
        out_shape=jax.ShapeDtypeStruct((P, M), f32),
    )(be, bv, xs, eW1, eb1.reshape(E, 1, M))

    h2 = pl.pallas_call(
        _ffn2_kernel,
        grid_spec=pltpu.PrefetchScalarGridSpec(
            num_scalar_prefetch=2,
            grid=(NB,),
            in_specs=[
                pl.BlockSpec((BT, M), lambda i, be, bv: (i, 0)),
                pl.BlockSpec((1, M, M), lambda i, be, bv: (be[i], 0, 0)),
                pl.BlockSpec((1, 1, M), lambda i, be, bv: (be[i], 0, 0)),
            ],
            out_specs=pl.BlockSpec((BT, M), lambda i, be, bv: (i, 0)),
        ),
        out_shape=jax.ShapeDtypeStruct((P, M), f32),
    )(be, bv, h1, eW2, eb2.reshape(E, 1, M))

    ys = pl.pallas_call(
        _ffn3_kernel,
        grid_spec=pltpu.PrefetchScalarGridSpec(
            num_scalar_prefetch=2,
            grid=(NB,),
            in_specs=[
                pl.BlockSpec((BT, M), lambda i, be, bv: (i, 0)),
                pl.BlockSpec((1, H, M), lambda i, be, bv: (be[i], 0, 0)),
                pl.BlockSpec((1, 1, H), lambda i, be, bv: (be[i], 0, 0)),
            ],
            out_specs=pl.BlockSpec((BT, H), lambda i, be, bv: (i, 0)),
        ),
        out_shape=jax.ShapeDtypeStruct((P, H), f32),
    )(be, bv, h2, eW3, eb3.reshape(E, 1, H))

    out = _make_combine(S, H, NC, NS)(ys, pos1, pos2, w1, w2)
    return out.reshape(b, s, h)


# speedup vs baseline: 1.8432x; 1.0595x over previous
import functools

import jax
import jax.numpy as jnp
from jax import lax
from jax.experimental import pallas as pl
from jax.experimental.pallas import tpu as pltpu
from jax.experimental.pallas import tpu_sc as plsc

EPS = 1e-6
TOPK = 2
BT = 256


def _dotT(a, b):
    return jax.lax.dot_general(a, b, (((1,), (1,)), ((), ())),
                               preferred_element_type=jnp.float32)


def _route_kernel(x_ref, ge_ref, rmsw_ref, wgW_ref, wgb_ref, ggW_ref, ggb_ref,
                  pos1_ref, pos2_ref, w1_ref, w2_ref, be_ref, bv_ref):
    S = x_ref.shape[0]
    NB = be_ref.shape[0]
    x = x_ref[...]
    var = jnp.mean(x * x, axis=-1, keepdims=True)
    xn = x * jax.lax.rsqrt(var + EPS) * rmsw_ref[...]
    gate = _dotT(xn, wgW_ref[...]) + wgb_ref[...]
    gate = gate + _dotT(ge_ref[...], ggW_ref[...]) + ggb_ref[...]
    gate = gate - jnp.max(gate, axis=-1, keepdims=True)
    eg = jnp.exp(gate)
    p = eg / jnp.sum(eg, axis=-1, keepdims=True)
    E = p.shape[-1]
    iota = jax.lax.broadcasted_iota(jnp.int32, p.shape, 1)
    w1 = jnp.max(p, axis=-1, keepdims=True)
    i1 = jnp.min(jnp.where(p == w1, iota, E), axis=-1, keepdims=True)
    oh1 = iota == i1
    pm = jnp.where(oh1, -jnp.inf, p)
    w2 = jnp.max(pm, axis=-1, keepdims=True)
    i2 = jnp.min(jnp.where(pm == w2, iota, E), axis=-1, keepdims=True)
    oh2 = iota == i2
    w1_ref[...] = jnp.broadcast_to(w1, (S, 16))
    w2_ref[...] = jnp.broadcast_to(w2, (S, 16))

    tot = (oh1 | oh2).astype(jnp.float32)
    r = jax.lax.broadcasted_iota(jnp.int32, (S, S), 0)
    c = jax.lax.broadcasted_iota(jnp.int32, (S, S), 1)
    L = (r > c).astype(jnp.float32)
    cum = jax.lax.dot_general(L, tot, (((1,), (0,)), ((), ())),
                              preferred_element_type=jnp.float32)
    counts = jnp.sum(tot, axis=0, keepdims=True)
    padded = jnp.ceil(counts / BT) * BT
    re = jax.lax.broadcasted_iota(jnp.int32, (E, E), 0)
    ce = jax.lax.broadcasted_iota(jnp.int32, (E, E), 1)
    offs = jax.lax.dot_general(padded, (re < ce).astype(jnp.float32),
                               (((1,), (0,)), ((), ())),
                               preferred_element_type=jnp.float32)
    pos1 = jnp.sum(jnp.where(oh1, offs + cum, 0.0), axis=1, keepdims=True)
    pos2 = jnp.sum(jnp.where(oh2, offs + cum, 0.0), axis=1, keepdims=True)
    pos1_ref[...] = pos1.astype(jnp.int32)
    pos2_ref[...] = pos2.astype(jnp.int32)

    si = jax.lax.broadcasted_iota(jnp.int32, (NB, E), 0).astype(jnp.float32) * BT
    ee = jax.lax.broadcasted_iota(jnp.int32, (NB, E), 1)
    inseg = (si >= offs) & (si < offs + padded)
    bv = jnp.sum(inseg.astype(jnp.int32), axis=1, keepdims=True)
    be = jnp.sum(jnp.where(inseg, ee, 0), axis=1, keepdims=True)
    be_ref[...] = jnp.where(bv > 0, be, E - 1)
    bv_ref[...] = bv


def _ffn1_kernel(be_ref, bv_ref, xs_ref, w_ref, b_ref, o_ref):
    i = pl.program_id(0)

    @pl.when(bv_ref[i] > 0)
    def _():
        a = xs_ref[...].astype(jnp.bfloat16)
        w = w_ref[0].astype(jnp.bfloat16)
        o_ref[...] = jnp.maximum(_dotT(a, w) + b_ref[0],
                                 0.0).astype(jnp.bfloat16)


def _ffn2_kernel(be_ref, bv_ref, h1_ref, w_ref, b_ref, o_ref):
    i = pl.program_id(0)

    @pl.when(bv_ref[i] > 0)
    def _():
        w = w_ref[0].astype(jnp.bfloat16)
        o_ref[...] = jnp.maximum(_dotT(h1_ref[...], w) + b_ref[0],
                                 0.0).astype(jnp.bfloat16)


def _ffn3_kernel(be_ref, bv_ref, h2_ref, w_ref, b_ref, o_ref):
    i = pl.program_id(0)

    @pl.when(bv_ref[i] > 0)
    def _():
        w = w_ref[0].astype(jnp.bfloat16)
        o_ref[...] = _dotT(h2_ref[...], w) + b_ref[0]


def _make_dispatch(S, H, P, NC, NS):
    NW = NC * NS
    CHUNK = S // NW
    mesh = plsc.VectorSubcoreMesh(core_axis_name="c", subcore_axis_name="s")

    @functools.partial(
        pl.kernel, mesh=mesh,
        out_type=jax.ShapeDtypeStruct((P, H), jnp.float32),
        scratch_types=[
            pltpu.VMEM((CHUNK,), jnp.int32),
            pltpu.VMEM((CHUNK,), jnp.int32),
            pltpu.VMEM((CHUNK, H), jnp.float32),
            pltpu.SemaphoreType.DMA,
        ],
    )
    def dispatch(x_hbm, pos1_hbm, pos2_hbm, xs_hbm, idx1_v, idx2_v, rows_v,
                 sem):
        wid = lax.axis_index("s") * NC + lax.axis_index("c")
        base = wid * CHUNK
        pltpu.sync_copy(pos1_hbm.at[pl.ds(base, CHUNK)], idx1_v)
        pltpu.sync_copy(pos2_hbm.at[pl.ds(base, CHUNK)], idx2_v)
        pltpu.sync_copy(x_hbm.at[pl.ds(base, CHUNK)], rows_v)
        pltpu.async_copy(rows_v, xs_hbm.at[idx1_v], sem).wait()
        pltpu.async_copy(rows_v, xs_hbm.at[idx2_v], sem).wait()

    return dispatch


def _make_combine(S, H, NC, NS):
    NW = NC * NS
    NSUB = 2
    CH = S // NW // NSUB
    NV = H // 16
    mesh = plsc.VectorSubcoreMesh(core_axis_name="c", subcore_axis_name="s")

    @functools.partial(
        pl.kernel, mesh=mesh,
        out_type=jax.ShapeDtypeStruct((S, H), jnp.float32),
        scratch_types=[
            pltpu.VMEM((CH,), jnp.int32),
            pltpu.VMEM((CH,), jnp.int32),
            pltpu.VMEM((CH, 16), jnp.float32),
            pltpu.VMEM((CH, 16), jnp.float32),
            pltpu.VMEM((CH, H), jnp.float32),
            pltpu.VMEM((CH, H), jnp.float32),
            pltpu.SemaphoreType.DMA,
        ],
    )
    def combine(ys_hbm, pos1_hbm, pos2_hbm, w1_hbm, w2_hbm, out_hbm,
                i1_v, i2_v, w1_v, w2_v, r1_v, r2_v, sem):
        wid = lax.axis_index("s") * NC + lax.axis_index("c")
        for cidx in range(NSUB):
            base = (wid * NSUB + cidx) * CH
            pltpu.sync_copy(pos1_hbm.at[pl.ds(base, CH)], i1_v)
            pltpu.sync_copy(pos2_hbm.at[pl.ds(base, CH)], i2_v)
            pltpu.sync_copy(w1_hbm.at[pl.ds(base, CH)], w1_v)
            pltpu.sync_copy(w2_hbm.at[pl.ds(base, CH)], w2_v)
            pltpu.async_copy(ys_hbm.at[i1_v], r1_v, sem).wait()
            pltpu.async_copy(ys_hbm.at[i2_v], r2_v, sem).wait()

            def body(i, _):
                w1s = w1_v[i, :]
                w2s = w2_v[i, :]
                for v in range(NV):
                    sl = pl.ds(v * 16, 16)
                    r1_v[i, sl] = r1_v[i, sl] * w1s + r2_v[i, sl] * w2s
                return 0

            lax.fori_loop(0, CH, body, 0)
            pltpu.sync_copy(r1_v, out_hbm.at[pl.ds(base, CH)])

    return combine


def kernel(x, genre_embed, rms_w, wg_W, wg_b, gg_W, gg_b,
           eW1, eb1, eW2, eb2, eW3, eb3):
    b, s, h = x.shape
    S = b * s
    E, M, H = eW1.shape
    G = gg_W.shape[1]
    P = TOPK * S + E * BT
    NB = P // BT
    x2 = x.reshape(S, H)
    ge2 = genre_embed.reshape(S, G)

    f32 = jnp.float32
    i32 = jnp.int32
    pos1, pos2, w1, w2, be, bv = pl.pallas_call(
        _route_kernel,
        out_shape=(
            jax.ShapeDtypeStruct((S, 1), i32),
            jax.ShapeDtypeStruct((S, 1), i32),
            jax.ShapeDtypeStruct((S, 16), f32),
            jax.ShapeDtypeStruct((S, 16), f32),
            jax.ShapeDtypeStruct((NB, 1), i32),
            jax.ShapeDtypeStruct((NB, 1), i32),
        ),
    )(x2, ge2, rms_w.reshape(1, H), wg_W, wg_b.reshape(1, E),
      gg_W, gg_b.reshape(1, E))
    pos1 = pos1.reshape(S)
    pos2 = pos2.reshape(S)
    be = be.reshape(NB)
    bv = bv.reshape(NB)

    info = plsc.get_sparse_core_info()
    NC, NS = info.num_cores, info.num_subcores

    xs = _make_dispatch(S, H, P, NC, NS)(x2, pos1, pos2)

    h1 = pl.pallas_call(
        _ffn1_kernel,
        grid_spec=pltpu.PrefetchScalarGridSpec(
            num_scalar_prefetch=2,
            grid=(NB,),
            in_specs=[
                pl.BlockSpec((BT, H), lambda i, be, bv: (i, 0)),
                pl.BlockSpec((1, M, H), lambda i, be, bv: (be[i], 0, 0)),
                pl.BlockSpec((1, 1, M), lambda i, be, bv: (be[i], 0, 0)),
            ],
            out_specs=pl.BlockSpec((BT, M), lambda i, be, bv: (i, 0)),
        ),
        out_shape=jax.ShapeDtypeStruct((P, M), jnp.bfloat16),
    )(be, bv, xs, eW1, eb1.reshape(E, 1, M))

    h2 = pl.pallas_call(
        _ffn2_kernel,
        grid_spec=pltpu.PrefetchScalarGridSpec(
            num_scalar_prefetch=2,
            grid=(NB,),
            in_specs=[
                pl.BlockSpec((BT, M), lambda i, be, bv: (i, 0)),
                pl.BlockSpec((1, M, M), lambda i, be, bv: (be[i], 0, 0)),
                pl.BlockSpec((1, 1, M), lambda i, be, bv: (be[i], 0, 0)),
            ],
            out_specs=pl.BlockSpec((BT, M), lambda i, be, bv: (i, 0)),
        ),
        out_shape=jax.ShapeDtypeStruct((P, M), jnp.bfloat16),
    )(be, bv, h1, eW2, eb2.reshape(E, 1, M))

    ys = pl.pallas_call(
        _ffn3_kernel,
        grid_spec=pltpu.PrefetchScalarGridSpec(
            num_scalar_prefetch=2,
            grid=(NB,),
            in_specs=[
                pl.BlockSpec((BT, M), lambda i, be, bv: (i, 0)),
                pl.BlockSpec((1, H, M), lambda i, be, bv: (be[i], 0, 0)),
                pl.BlockSpec((1, 1, H), lambda i, be, bv: (be[i], 0, 0)),
            ],
            out_specs=pl.BlockSpec((BT, H), lambda i, be, bv: (i, 0)),
        ),
        out_shape=jax.ShapeDtypeStruct((P, H), f32),
    )(be, bv, h2, eW3, eb3.reshape(E, 1, H))

    out = _make_combine(S, H, NC, NS)(ys, pos1, pos2, w1, w2)
    return out.reshape(b, s, h)


# speedup vs baseline: 2.0144x; 1.0929x over previous
import functools

import jax
import jax.numpy as jnp
from jax import lax
from jax.experimental import pallas as pl
from jax.experimental.pallas import tpu as pltpu
from jax.experimental.pallas import tpu_sc as plsc

EPS = 1e-6
TOPK = 2
BT = 256


def _dotT(a, b):
    return jax.lax.dot_general(a, b, (((1,), (1,)), ((), ())),
                               preferred_element_type=jnp.float32)


def _route_kernel(x_ref, ge_ref, rmsw_ref, wgW_ref, wgb_ref, ggW_ref, ggb_ref,
                  pos1_ref, pos2_ref, w1_ref, w2_ref, be_ref, bv_ref):
    S = x_ref.shape[0]
    NB = be_ref.shape[0]
    x = x_ref[...]
    var = jnp.mean(x * x, axis=-1, keepdims=True)
    xn = x * jax.lax.rsqrt(var + EPS) * rmsw_ref[...]
    gate = _dotT(xn, wgW_ref[...]) + wgb_ref[...]
    gate = gate + _dotT(ge_ref[...], ggW_ref[...]) + ggb_ref[...]
    gate = gate - jnp.max(gate, axis=-1, keepdims=True)
    eg = jnp.exp(gate)
    p = eg / jnp.sum(eg, axis=-1, keepdims=True)
    E = p.shape[-1]
    iota = jax.lax.broadcasted_iota(jnp.int32, p.shape, 1)
    w1 = jnp.max(p, axis=-1, keepdims=True)
    i1 = jnp.min(jnp.where(p == w1, iota, E), axis=-1, keepdims=True)
    oh1 = iota == i1
    pm = jnp.where(oh1, -jnp.inf, p)
    w2 = jnp.max(pm, axis=-1, keepdims=True)
    i2 = jnp.min(jnp.where(pm == w2, iota, E), axis=-1, keepdims=True)
    oh2 = iota == i2
    w1_ref[...] = jnp.broadcast_to(w1, (S, 16))
    w2_ref[...] = jnp.broadcast_to(w2, (S, 16))

    tot = (oh1 | oh2).astype(jnp.float32)
    r = jax.lax.broadcasted_iota(jnp.int32, (S, S), 0)
    c = jax.lax.broadcasted_iota(jnp.int32, (S, S), 1)
    L = (r > c).astype(jnp.float32)
    cum = jax.lax.dot_general(L, tot, (((1,), (0,)), ((), ())),
                              preferred_element_type=jnp.float32)
    counts = jnp.sum(tot, axis=0, keepdims=True)
    padded = jnp.ceil(counts / BT) * BT
    re = jax.lax.broadcasted_iota(jnp.int32, (E, E), 0)
    ce = jax.lax.broadcasted_iota(jnp.int32, (E, E), 1)
    offs = jax.lax.dot_general(padded, (re < ce).astype(jnp.float32),
                               (((1,), (0,)), ((), ())),
                               preferred_element_type=jnp.float32)
    pos1 = jnp.sum(jnp.where(oh1, offs + cum, 0.0), axis=1, keepdims=True)
    pos2 = jnp.sum(jnp.where(oh2, offs + cum, 0.0), axis=1, keepdims=True)
    pos1_ref[...] = pos1.astype(jnp.int32)
    pos2_ref[...] = pos2.astype(jnp.int32)

    si = jax.lax.broadcasted_iota(jnp.int32, (NB, E), 0).astype(jnp.float32) * BT
    ee = jax.lax.broadcasted_iota(jnp.int32, (NB, E), 1)
    inseg = (si >= offs) & (si < offs + padded)
    bv = jnp.sum(inseg.astype(jnp.int32), axis=1, keepdims=True)
    be = jnp.sum(jnp.where(inseg, ee, 0), axis=1, keepdims=True)
    be_ref[...] = jnp.where(bv > 0, be, E - 1)
    bv_ref[...] = bv


def _ffn1_kernel(be_ref, bv_ref, xs_ref, w_ref, b_ref, o_ref):
    i = pl.program_id(0)

    @pl.when(bv_ref[i] > 0)
    def _():
        a = xs_ref[...].astype(jnp.bfloat16)
        w = w_ref[0].astype(jnp.bfloat16)
        o_ref[...] = jnp.maximum(_dotT(a, w) + b_ref[0],
                                 0.0).astype(jnp.bfloat16)


def _ffn23_kernel(be_ref, bv_ref, h1_ref, w2_ref, b2_ref, w3_ref, b3_ref,
                  o_ref):
    i = pl.program_id(0)

    @pl.when(bv_ref[i] > 0)
    def _():
        w2 = w2_ref[0].astype(jnp.bfloat16)
        h2 = jnp.maximum(_dotT(h1_ref[...], w2) + b2_ref[0],
                         0.0).astype(jnp.bfloat16)
        w3 = w3_ref[0].astype(jnp.bfloat16)
        o_ref[...] = _dotT(h2, w3) + b3_ref[0]


def _make_dispatch(S, H, P, NC, NS):
    NW = NC * NS
    CHUNK = S // NW
    mesh = plsc.VectorSubcoreMesh(core_axis_name="c", subcore_axis_name="s")

    @functools.partial(
        pl.kernel, mesh=mesh,
        out_type=jax.ShapeDtypeStruct((P, H), jnp.float32),
        scratch_types=[
            pltpu.VMEM((CHUNK,), jnp.int32),
            pltpu.VMEM((CHUNK,), jnp.int32),
            pltpu.VMEM((CHUNK, H), jnp.float32),
            pltpu.SemaphoreType.DMA,
        ],
    )
    def dispatch(x_hbm, pos1_hbm, pos2_hbm, xs_hbm, idx1_v, idx2_v, rows_v,
                 sem):
        wid = lax.axis_index("s") * NC + lax.axis_index("c")
        base = wid * CHUNK
        pltpu.sync_copy(pos1_hbm.at[pl.ds(base, CHUNK)], idx1_v)
        pltpu.sync_copy(pos2_hbm.at[pl.ds(base, CHUNK)], idx2_v)
        pltpu.sync_copy(x_hbm.at[pl.ds(base, CHUNK)], rows_v)
        pltpu.async_copy(rows_v, xs_hbm.at[idx1_v], sem).wait()
        pltpu.async_copy(rows_v, xs_hbm.at[idx2_v], sem).wait()

    return dispatch


def _make_combine(S, H, NC, NS):
    NW = NC * NS
    NSUB = 2
    CH = S // NW // NSUB
    NV = H // 16
    mesh = plsc.VectorSubcoreMesh(core_axis_name="c", subcore_axis_name="s")

    @functools.partial(
        pl.kernel, mesh=mesh,
        out_type=jax.ShapeDtypeStruct((S, H), jnp.float32),
        scratch_types=[
            pltpu.VMEM((CH,), jnp.int32),
            pltpu.VMEM((CH,), jnp.int32),
            pltpu.VMEM((CH, 16), jnp.float32),
            pltpu.VMEM((CH, 16), jnp.float32),
            pltpu.VMEM((CH, H), jnp.float32),
            pltpu.VMEM((CH, H), jnp.float32),
            pltpu.SemaphoreType.DMA,
        ],
    )
    def combine(ys_hbm, pos1_hbm, pos2_hbm, w1_hbm, w2_hbm, out_hbm,
                i1_v, i2_v, w1_v, w2_v, r1_v, r2_v, sem):
        wid = lax.axis_index("s") * NC + lax.axis_index("c")
        for cidx in range(NSUB):
            base = (wid * NSUB + cidx) * CH
            pltpu.sync_copy(pos1_hbm.at[pl.ds(base, CH)], i1_v)
            pltpu.sync_copy(pos2_hbm.at[pl.ds(base, CH)], i2_v)
            pltpu.sync_copy(w1_hbm.at[pl.ds(base, CH)], w1_v)
            pltpu.sync_copy(w2_hbm.at[pl.ds(base, CH)], w2_v)
            pltpu.async_copy(ys_hbm.at[i1_v], r1_v, sem).wait()
            pltpu.async_copy(ys_hbm.at[i2_v], r2_v, sem).wait()

            def body(i, _):
                w1s = w1_v[i, :]
                w2s = w2_v[i, :]
                for v in range(NV):
                    sl = pl.ds(v * 16, 16)
                    r1_v[i, sl] = r1_v[i, sl] * w1s + r2_v[i, sl] * w2s
                return 0

            lax.fori_loop(0, CH, body, 0)
            pltpu.sync_copy(r1_v, out_hbm.at[pl.ds(base, CH)])

    return combine


def kernel(x, genre_embed, rms_w, wg_W, wg_b, gg_W, gg_b,
           eW1, eb1, eW2, eb2, eW3, eb3):
    b, s, h = x.shape
    S = b * s
    E, M, H = eW1.shape
    G = gg_W.shape[1]
    P = TOPK * S + E * BT
    NB = P // BT
    x2 = x.reshape(S, H)
    ge2 = genre_embed.reshape(S, G)

    f32 = jnp.float32
    i32 = jnp.int32
    pos1, pos2, w1, w2, be, bv = pl.pallas_call(
        _route_kernel,
        out_shape=(
            jax.ShapeDtypeStruct((S, 1), i32),
            jax.ShapeDtypeStruct((S, 1), i32),
            jax.ShapeDtypeStruct((S, 16), f32),
            jax.ShapeDtypeStruct((S, 16), f32),
            jax.ShapeDtypeStruct((NB, 1), i32),
            jax.ShapeDtypeStruct((NB, 1), i32),
        ),
    )(x2, ge2, rms_w.reshape(1, H), wg_W, wg_b.reshape(1, E),
      gg_W, gg_b.reshape(1, E))
    pos1 = pos1.reshape(S)
    pos2 = pos2.reshape(S)
    be = be.reshape(NB)
    bv = bv.reshape(NB)

    info = plsc.get_sparse_core_info()
    NC, NS = info.num_cores, info.num_subcores

    xs = _make_dispatch(S, H, P, NC, NS)(x2, pos1, pos2)

    h1 = pl.pallas_call(
        _ffn1_kernel,
        grid_spec=pltpu.PrefetchScalarGridSpec(
            num_scalar_prefetch=2,
            grid=(NB,),
            in_specs=[
                pl.BlockSpec((BT, H), lambda i, be, bv: (i, 0)),
                pl.BlockSpec((1, M, H), lambda i, be, bv: (be[i], 0, 0)),
                pl.BlockSpec((1, 1, M), lambda i, be, bv: (be[i], 0, 0)),
            ],
            out_specs=pl.BlockSpec((BT, M), lambda i, be, bv: (i, 0)),
        ),
        out_shape=jax.ShapeDtypeStruct((P, M), jnp.bfloat16),
    )(be, bv, xs, eW1, eb1.reshape(E, 1, M))

    ys = pl.pallas_call(
        _ffn23_kernel,
        grid_spec=pltpu.PrefetchScalarGridSpec(
            num_scalar_prefetch=2,
            grid=(NB,),
            in_specs=[
                pl.BlockSpec((BT, M), lambda i, be, bv: (i, 0)),
                pl.BlockSpec((1, M, M), lambda i, be, bv: (be[i], 0, 0)),
                pl.BlockSpec((1, 1, M), lambda i, be, bv: (be[i], 0, 0)),
                pl.BlockSpec((1, H, M), lambda i, be, bv: (be[i], 0, 0)),
                pl.BlockSpec((1, 1, H), lambda i, be, bv: (be[i], 0, 0)),
            ],
            out_specs=pl.BlockSpec((BT, H), lambda i, be, bv: (i, 0)),
        ),
        out_shape=jax.ShapeDtypeStruct((P, H), f32),
    )(be, bv, h1, eW2, eb2.reshape(E, 1, M), eW3, eb3.reshape(E, 1, H))

    out = _make_combine(S, H, NC, NS)(ys, pos1, pos2, w1, w2)
    return out.reshape(b, s, h)
